# baseline (speedup 1.0000x reference)
import functools

import jax
import jax.numpy as jnp
from jax import lax
from jax.experimental import pallas as pl
from jax.experimental.pallas import tpu as pltpu
from jax.experimental.pallas import tpu_sc as plsc

N, M, F, D_E, H = 10000, 32, 128, 16, 128
BN_BLK = 200
NBLK = N // BN_BLK
MJ = 2
NJ = M // MJ

def _q_kernel(atom_ref, w1b_ref, q_ref):
    q_ref[...] = jnp.dot(atom_ref[...], w1b_ref[...],
                         preferred_element_type=jnp.float32)


def _compute_q(atom, w1b):
    return pl.pallas_call(
        _q_kernel,
        grid=(NBLK,),
        in_specs=[
            pl.BlockSpec((BN_BLK, F), lambda i: (i, 0)),
            pl.BlockSpec((F, H), lambda i: (0, 0)),
        ],
        out_specs=pl.BlockSpec((BN_BLK, H), lambda i: (i, 0)),
        out_shape=jax.ShapeDtypeStruct((N, H), jnp.float32),
    )(atom, w1b)


_NC, _NS = 2, 16
_NW = _NC * _NS
_E_PER_W = (N * M) // _NW
_CHUNK = 400
_NCHUNK = _E_PER_W // _CHUNK


def _gather_body(q_hbm, idx_hbm, out_hbm, idx_v, rows_v, sem):
    wid = lax.axis_index("s") * _NC + lax.axis_index("c")
    base = wid * _E_PER_W

    def step(c, _):
        off = base + c * _CHUNK
        pltpu.sync_copy(idx_hbm.at[pl.ds(off, _CHUNK)], idx_v)
        pltpu.async_copy(q_hbm.at[idx_v], rows_v, sem).wait()
        pltpu.sync_copy(rows_v, out_hbm.at[pl.ds(off, _CHUNK)])
        return ()

    lax.fori_loop(0, _NCHUNK, step, (), unroll=False)


def _gather_rows(q, idx_flat):
    mesh = plsc.VectorSubcoreMesh(core_axis_name="c", subcore_axis_name="s")
    fn = functools.partial(
        pl.kernel, mesh=mesh,
        out_type=jax.ShapeDtypeStruct((N * M, H), jnp.float32),
        scratch_types=[
            pltpu.VMEM((_CHUNK,), jnp.int32),
            pltpu.VMEM((_CHUNK, H), jnp.float32),
            pltpu.SemaphoreType.DMA,
        ],
    )(_gather_body)
    return fn(q, idx_flat)


_LOG2E = 1.4426950408889634
_LN2 = 0.6931471805599453


def _softplus(x):
    t = lax.exp2(-jnp.abs(x) * _LOG2E)
    return jnp.maximum(x, 0.0) + jnp.log(1.0 + t)


def _msg_kernel(atom_ref, qg_ref, nbr_ref, idxf_ref, w1a_ref, w1c_ref,
                b1_ref, w2_ref, b2_ref, out_ref, psum_ref, psumsq_ref,
                p_sc, acc_sc, cnt_sc):
    j = pl.program_id(1)

    @pl.when(j == 0)
    def _init():
        p_sc[...] = jnp.dot(atom_ref[...], w1a_ref[...],
                            preferred_element_type=jnp.float32) + b1_ref[...]
        acc_sc[...] = jnp.zeros_like(acc_sc)
        cnt_sc[...] = jnp.zeros_like(cnt_sc)

    p = p_sc[...]
    for jj in range(MJ):
        e2 = jnp.dot(nbr_ref[jj], w1c_ref[...],
                     preferred_element_type=jnp.float32)
        a = qg_ref[jj] + e2 + p
        h = _softplus(a)
        m = idxf_ref[jj] != 0.0
        acc_sc[...] += jnp.where(m, h, 0.0)
        cnt_sc[...] += jnp.where(m, 1.0, 0.0)

    @pl.when(j == NJ - 1)
    def _fin():
        atom = atom_ref[...]
        msg = (jnp.dot(acc_sc[...], w2_ref[...],
                       preferred_element_type=jnp.float32)
               + cnt_sc[...] * b2_ref[...])
        out_pre = atom + msg
        out_ref[...] = out_pre
        psum_ref[...] = jnp.sum(out_pre, axis=0, keepdims=True)[None]
        psumsq_ref[...] = jnp.sum(out_pre * out_pre, axis=0,
                                  keepdims=True)[None]


def _compute_msg(atom, qg3, nbr_t, idxf_t, w1a, w1c, b1r, w2, b2r):
    return pl.pallas_call(
        _msg_kernel,
        grid=(NBLK, NJ),
        in_specs=[
            pl.BlockSpec((BN_BLK, F), lambda i, j: (i, 0)),
            pl.BlockSpec((MJ, BN_BLK, H), lambda i, j: (j, i, 0)),
            pl.BlockSpec((MJ, BN_BLK, D_E), lambda i, j: (j, i, 0)),
            pl.BlockSpec((MJ, BN_BLK, 1), lambda i, j: (j, i, 0)),
            pl.BlockSpec((F, H), lambda i, j: (0, 0)),
            pl.BlockSpec((D_E, H), lambda i, j: (0, 0)),
            pl.BlockSpec((1, H), lambda i, j: (0, 0)),
            pl.BlockSpec((H, F), lambda i, j: (0, 0)),
            pl.BlockSpec((1, F), lambda i, j: (0, 0)),
        ],
        out_specs=[
            pl.BlockSpec((BN_BLK, F), lambda i, j: (i, 0)),
            pl.BlockSpec((1, 1, F), lambda i, j: (i, 0, 0)),
            pl.BlockSpec((1, 1, F), lambda i, j: (i, 0, 0)),
        ],
        out_shape=[
            jax.ShapeDtypeStruct((N, F), jnp.float32),
            jax.ShapeDtypeStruct((NBLK, 1, F), jnp.float32),
            jax.ShapeDtypeStruct((NBLK, 1, F), jnp.float32),
        ],
        scratch_shapes=[
            pltpu.VMEM((BN_BLK, H), jnp.float32),
            pltpu.VMEM((BN_BLK, H), jnp.float32),
            pltpu.VMEM((BN_BLK, 1), jnp.float32),
        ],
    )(atom, qg3, nbr_t, idxf_t, w1a, w1c, b1r, w2, b2r)


def _bn_kernel(x_ref, psum_ref, psumsq_ref, gamma_ref, beta_ref, out_ref):
    mean = jnp.sum(psum_ref[...], axis=0) / N
    ex2 = jnp.sum(psumsq_ref[...], axis=0) / N
    var = ex2 - mean * mean
    inv = lax.rsqrt(var + 1e-5)
    y = (x_ref[...] - mean) * (inv * gamma_ref[...]) + beta_ref[...]
    out_ref[...] = _softplus(y)


def _apply_bn(x, psum, psumsq, gammar, betar):
    return pl.pallas_call(
        _bn_kernel,
        grid=(NBLK,),
        in_specs=[
            pl.BlockSpec((BN_BLK, F), lambda i: (i, 0)),
            pl.BlockSpec((NBLK, 1, F), lambda i: (0, 0, 0)),
            pl.BlockSpec((NBLK, 1, F), lambda i: (0, 0, 0)),
            pl.BlockSpec((1, F), lambda i: (0, 0)),
            pl.BlockSpec((1, F), lambda i: (0, 0)),
        ],
        out_specs=pl.BlockSpec((BN_BLK, F), lambda i: (i, 0)),
        out_shape=jax.ShapeDtypeStruct((N, F), jnp.float32),
    )(x, psum, psumsq, gammar, betar)


def kernel(atom_in_fea, nbr_fea, nbr_fea_idx, W1, b1, W2, b2,
           bn_gamma, bn_beta):
    w1a = W1[:F]
    w1b = W1[F:2 * F]
    w1c = W1[2 * F:]
    b1r = b1.reshape(1, H)
    b2r = b2.reshape(1, F)
    gammar = bn_gamma.reshape(1, F)
    betar = bn_beta.reshape(1, F)
    idx_t = nbr_fea_idx.T
    idx_flat_t = idx_t.reshape(N * M)
    idxf_t = idx_t.astype(jnp.float32).reshape(M, N, 1)
    nbr_t = jnp.transpose(nbr_fea, (1, 0, 2))

    q = _compute_q(atom_in_fea, w1b)
    qg3 = _gather_rows(q, idx_flat_t).reshape(M, N, H)
    out_pre, psum, psumsq = _compute_msg(
        atom_in_fea, qg3, nbr_t, idxf_t, w1a, w1c, b1r, W2, b2r)
    return _apply_bn(out_pre, psum, psumsq, gammar, betar)


# speedup vs baseline: 1.7766x; 1.7766x over previous
import functools

import jax
import jax.numpy as jnp
from jax import lax
from jax.experimental import pallas as pl
from jax.experimental.pallas import tpu as pltpu
from jax.experimental.pallas import tpu_sc as plsc

N, M, F, D_E, H = 10000, 32, 128, 16, 128
BN_BLK = 400
NBLK = N // BN_BLK

def _q_kernel(atom_ref, w1b_ref, q_ref):
    q_ref[...] = jnp.dot(atom_ref[...], w1b_ref[...],
                         preferred_element_type=jnp.float32)


def _compute_q(atom, w1b):
    return pl.pallas_call(
        _q_kernel,
        grid=(NBLK,),
        in_specs=[
            pl.BlockSpec((BN_BLK, F), lambda i: (i, 0)),
            pl.BlockSpec((F, H), lambda i: (0, 0)),
        ],
        out_specs=pl.BlockSpec((BN_BLK, H), lambda i: (i, 0)),
        out_shape=jax.ShapeDtypeStruct((N, H), jnp.float32),
    )(atom, w1b)


_NC, _NS = 2, 16
_NW = _NC * _NS
_E_PER_W = (N * M) // _NW
_CHUNK = 400
_NCHUNK = _E_PER_W // _CHUNK


def _gather_body(q_hbm, idx_hbm, out_hbm, idx_v, rows_v, sem):
    wid = lax.axis_index("s") * _NC + lax.axis_index("c")
    base = wid * _E_PER_W

    def step(c, _):
        off = base + c * _CHUNK
        pltpu.sync_copy(idx_hbm.at[pl.ds(off, _CHUNK)], idx_v)
        pltpu.async_copy(q_hbm.at[idx_v], rows_v, sem).wait()
        pltpu.sync_copy(rows_v, out_hbm.at[pl.ds(off, _CHUNK)])
        return ()

    lax.fori_loop(0, _NCHUNK, step, (), unroll=False)


def _gather_rows(q, idx_flat):
    mesh = plsc.VectorSubcoreMesh(core_axis_name="c", subcore_axis_name="s")
    fn = functools.partial(
        pl.kernel, mesh=mesh,
        out_type=jax.ShapeDtypeStruct((N * M, H), jnp.float32),
        scratch_types=[
            pltpu.VMEM((_CHUNK,), jnp.int32),
            pltpu.VMEM((_CHUNK, H), jnp.float32),
            pltpu.SemaphoreType.DMA,
        ],
    )(_gather_body)
    return fn(q, idx_flat)


_LOG2E = 1.4426950408889634


def _softplus(x):
    t = lax.exp2(-jnp.abs(x) * _LOG2E)
    return jnp.maximum(x, 0.0) + jnp.log(1.0 + t)


def _msg_kernel(atom_ref, qg_ref, nbr_ref, idxf_ref, w1a_ref, w1c_ref,
                b1_ref, w2_ref, b2_ref, out_ref, psum_ref, psumsq_ref):
    atom = atom_ref[...]
    p = jnp.dot(atom, w1a_ref[...],
                preferred_element_type=jnp.float32) + b1_ref[...]
    nbr2 = nbr_ref[...].reshape(BN_BLK * M, D_E)
    e2 = jnp.dot(nbr2, w1c_ref[...],
                 preferred_element_type=jnp.float32)
    a3 = qg_ref[...] + e2.reshape(BN_BLK, M, H) + p[:, None, :]
    h3 = _softplus(a3)
    m3 = idxf_ref[...] != 0.0
    hsum = jnp.sum(jnp.where(m3, h3, 0.0), axis=1)
    cnt = jnp.sum(jnp.where(m3, 1.0, 0.0), axis=1)
    msg = (jnp.dot(hsum, w2_ref[...], preferred_element_type=jnp.float32)
           + cnt * b2_ref[...])
    out_pre = atom + msg
    out_ref[...] = out_pre
    psum_ref[...] = jnp.sum(out_pre, axis=0, keepdims=True)[None]
    psumsq_ref[...] = jnp.sum(out_pre * out_pre, axis=0, keepdims=True)[None]


def _compute_msg(atom, qg3, nbr_fea, idxf3, w1a, w1c, b1r, w2, b2r):
    return pl.pallas_call(
        _msg_kernel,
        grid=(NBLK,),
        in_specs=[
            pl.BlockSpec((BN_BLK, F), lambda i: (i, 0)),
            pl.BlockSpec((BN_BLK, M, H), lambda i: (i, 0, 0)),
            pl.BlockSpec((BN_BLK, M, D_E), lambda i: (i, 0, 0)),
            pl.BlockSpec((BN_BLK, M, 1), lambda i: (i, 0, 0)),
            pl.BlockSpec((F, H), lambda i: (0, 0)),
            pl.BlockSpec((D_E, H), lambda i: (0, 0)),
            pl.BlockSpec((1, H), lambda i: (0, 0)),
            pl.BlockSpec((H, F), lambda i: (0, 0)),
            pl.BlockSpec((1, F), lambda i: (0, 0)),
        ],
        out_specs=[
            pl.BlockSpec((BN_BLK, F), lambda i: (i, 0)),
            pl.BlockSpec((1, 1, F), lambda i: (i, 0, 0)),
            pl.BlockSpec((1, 1, F), lambda i: (i, 0, 0)),
        ],
        out_shape=[
            jax.ShapeDtypeStruct((N, F), jnp.float32),
            jax.ShapeDtypeStruct((NBLK, 1, F), jnp.float32),
            jax.ShapeDtypeStruct((NBLK, 1, F), jnp.float32),
        ],
    )(atom, qg3, nbr_fea, idxf3, w1a, w1c, b1r, w2, b2r)


def _bn_kernel(x_ref, psum_ref, psumsq_ref, gamma_ref, beta_ref, out_ref):
    mean = jnp.sum(psum_ref[...], axis=0) / N
    ex2 = jnp.sum(psumsq_ref[...], axis=0) / N
    var = ex2 - mean * mean
    inv = lax.rsqrt(var + 1e-5)
    y = (x_ref[...] - mean) * (inv * gamma_ref[...]) + beta_ref[...]
    out_ref[...] = _softplus(y)


def _apply_bn(x, psum, psumsq, gammar, betar):
    return pl.pallas_call(
        _bn_kernel,
        grid=(NBLK,),
        in_specs=[
            pl.BlockSpec((BN_BLK, F), lambda i: (i, 0)),
            pl.BlockSpec((NBLK, 1, F), lambda i: (0, 0, 0)),
            pl.BlockSpec((NBLK, 1, F), lambda i: (0, 0, 0)),
            pl.BlockSpec((1, F), lambda i: (0, 0)),
            pl.BlockSpec((1, F), lambda i: (0, 0)),
        ],
        out_specs=pl.BlockSpec((BN_BLK, F), lambda i: (i, 0)),
        out_shape=jax.ShapeDtypeStruct((N, F), jnp.float32),
    )(x, psum, psumsq, gammar, betar)


def kernel(atom_in_fea, nbr_fea, nbr_fea_idx, W1, b1, W2, b2,
           bn_gamma, bn_beta):
    w1a = W1[:F]
    w1b = W1[F:2 * F]
    w1c = W1[2 * F:]
    b1r = b1.reshape(1, H)
    b2r = b2.reshape(1, F)
    gammar = bn_gamma.reshape(1, F)
    betar = bn_beta.reshape(1, F)
    idx_flat = nbr_fea_idx.reshape(N * M)
    idxf3 = nbr_fea_idx.astype(jnp.float32).reshape(N, M, 1)

    q = _compute_q(atom_in_fea, w1b)
    qg = _gather_rows(q, idx_flat).reshape(N, M, H)
    out_pre, psum, psumsq = _compute_msg(
        atom_in_fea, qg, nbr_fea, idxf3, w1a, w1c, b1r, W2, b2r)
    return _apply_bn(out_pre, psum, psumsq, gammar, betar)


# speedup vs baseline: 2.8452x; 1.6015x over previous
import functools

import jax
import jax.numpy as jnp
from jax import lax
from jax.experimental import pallas as pl
from jax.experimental.pallas import tpu as pltpu
from jax.experimental.pallas import tpu_sc as plsc

N, M, F, D_E, H = 10000, 32, 128, 16, 128
BN_BLK = 400
NBLK = N // BN_BLK

def _q_kernel(atom_ref, w1b_ref, q_ref):
    q = jnp.dot(atom_ref[...], w1b_ref[...],
                preferred_element_type=jnp.float32)
    rid = (lax.broadcasted_iota(jnp.int32, (BN_BLK, H), 0)
           + pl.program_id(0) * BN_BLK)
    q_ref[...] = jnp.where(rid == 0, -1e9, q)


def _compute_q(atom, w1b):
    return pl.pallas_call(
        _q_kernel,
        grid=(NBLK,),
        in_specs=[
            pl.BlockSpec((BN_BLK, F), lambda i: (i, 0)),
            pl.BlockSpec((F, H), lambda i: (0, 0)),
        ],
        out_specs=pl.BlockSpec((BN_BLK, H), lambda i: (i, 0)),
        out_shape=jax.ShapeDtypeStruct((N, H), jnp.float32),
    )(atom, w1b)


_NC, _NS = 2, 16
_NW = _NC * _NS
_CHUNK = 400
_NCHUNK = N // _CHUNK


def _gather_body(q_hbm, idx_hbm, out_hbm, idx_v, rows_v, sem):
    wid = lax.axis_index("s") * _NC + lax.axis_index("c")
    base = wid * N

    def step(c, _):
        off = base + c * _CHUNK
        pltpu.sync_copy(idx_hbm.at[pl.ds(off, _CHUNK)], idx_v)
        pltpu.async_copy(q_hbm.at[idx_v], rows_v, sem).wait()
        pltpu.sync_copy(rows_v, out_hbm.at[pl.ds(off, _CHUNK)])
        return ()

    lax.fori_loop(0, _NCHUNK, step, (), unroll=False)


def _gather_rows(q, idx_flat_t):
    mesh = plsc.VectorSubcoreMesh(core_axis_name="c", subcore_axis_name="s")
    fn = functools.partial(
        pl.kernel, mesh=mesh,
        out_type=jax.ShapeDtypeStruct((M * N, H), jnp.float32),
        scratch_types=[
            pltpu.VMEM((_CHUNK,), jnp.int32),
            pltpu.VMEM((_CHUNK, H), jnp.float32),
            pltpu.SemaphoreType.DMA,
        ],
    )(_gather_body)
    return fn(q, idx_flat_t)


_LOG2E = 1.4426950408889634


def _softplus(x):
    t = lax.exp2(-jnp.abs(x) * _LOG2E)
    return jnp.maximum(x, 0.0) + jnp.log(1.0 + t)


def _msg_kernel(atom_ref, qg_ref, nbr_ref, idxf_ref, w1a_ref, w1c_ref,
                b1_ref, w2_ref, b2_ref, out_ref, psum_ref, psumsq_ref):
    atom = atom_ref[...]
    p = jnp.dot(atom, w1a_ref[...],
                preferred_element_type=jnp.float32) + b1_ref[...]
    w1c = w1c_ref[...]
    acc = None
    for j in range(M):
        ej = jnp.dot(nbr_ref[:, j * D_E:(j + 1) * D_E], w1c,
                     preferred_element_type=jnp.float32)
        hj = _softplus(qg_ref[j] + ej + p)
        acc = hj if acc is None else acc + hj
    cnt = jnp.sum(jnp.where(idxf_ref[...] != 0.0, 1.0, 0.0),
                  axis=1, keepdims=True)
    msg = (jnp.dot(acc, w2_ref[...], preferred_element_type=jnp.float32)
           + cnt * b2_ref[...])
    out_pre = atom + msg
    out_ref[...] = out_pre
    psum_ref[...] = jnp.sum(out_pre, axis=0, keepdims=True)[None]
    psumsq_ref[...] = jnp.sum(out_pre * out_pre, axis=0, keepdims=True)[None]


def _compute_msg(atom, qg3, nbr2, idxf2, w1a, w1c, b1r, w2, b2r):
    return pl.pallas_call(
        _msg_kernel,
        grid=(NBLK,),
        in_specs=[
            pl.BlockSpec((BN_BLK, F), lambda i: (i, 0)),
            pl.BlockSpec((M, BN_BLK, H), lambda i: (0, i, 0)),
            pl.BlockSpec((BN_BLK, M * D_E), lambda i: (i, 0)),
            pl.BlockSpec((BN_BLK, M), lambda i: (i, 0)),
            pl.BlockSpec((F, H), lambda i: (0, 0)),
            pl.BlockSpec((D_E, H), lambda i: (0, 0)),
            pl.BlockSpec((1, H), lambda i: (0, 0)),
            pl.BlockSpec((H, F), lambda i: (0, 0)),
            pl.BlockSpec((1, F), lambda i: (0, 0)),
        ],
        out_specs=[
            pl.BlockSpec((BN_BLK, F), lambda i: (i, 0)),
            pl.BlockSpec((1, 1, F), lambda i: (i, 0, 0)),
            pl.BlockSpec((1, 1, F), lambda i: (i, 0, 0)),
        ],
        out_shape=[
            jax.ShapeDtypeStruct((N, F), jnp.float32),
            jax.ShapeDtypeStruct((NBLK, 1, F), jnp.float32),
            jax.ShapeDtypeStruct((NBLK, 1, F), jnp.float32),
        ],
    )(atom, qg3, nbr2, idxf2, w1a, w1c, b1r, w2, b2r)


def _bn_kernel(x_ref, psum_ref, psumsq_ref, gamma_ref, beta_ref, out_ref):
    mean = jnp.sum(psum_ref[...], axis=0) / N
    ex2 = jnp.sum(psumsq_ref[...], axis=0) / N
    var = ex2 - mean * mean
    inv = lax.rsqrt(var + 1e-5)
    y = (x_ref[...] - mean) * (inv * gamma_ref[...]) + beta_ref[...]
    out_ref[...] = _softplus(y)


def _apply_bn(x, psum, psumsq, gammar, betar):
    return pl.pallas_call(
        _bn_kernel,
        grid=(NBLK,),
        in_specs=[
            pl.BlockSpec((BN_BLK, F), lambda i: (i, 0)),
            pl.BlockSpec((NBLK, 1, F), lambda i: (0, 0, 0)),
            pl.BlockSpec((NBLK, 1, F), lambda i: (0, 0, 0)),
            pl.BlockSpec((1, F), lambda i: (0, 0)),
            pl.BlockSpec((1, F), lambda i: (0, 0)),
        ],
        out_specs=pl.BlockSpec((BN_BLK, F), lambda i: (i, 0)),
        out_shape=jax.ShapeDtypeStruct((N, F), jnp.float32),
    )(x, psum, psumsq, gammar, betar)


def kernel(atom_in_fea, nbr_fea, nbr_fea_idx, W1, b1, W2, b2,
           bn_gamma, bn_beta):
    w1a = W1[:F]
    w1b = W1[F:2 * F]
    w1c = W1[2 * F:]
    b1r = b1.reshape(1, H)
    b2r = b2.reshape(1, F)
    gammar = bn_gamma.reshape(1, F)
    betar = bn_beta.reshape(1, F)
    nbr2 = nbr_fea.reshape(N, M * D_E)
    idxf2 = nbr_fea_idx.astype(jnp.float32)

    idx_flat_t = nbr_fea_idx.T.reshape(M * N)
    q = _compute_q(atom_in_fea, w1b)
    qg3 = _gather_rows(q, idx_flat_t).reshape(M, N, H)
    out_pre, psum, psumsq = _compute_msg(
        atom_in_fea, qg3, nbr2, idxf2, w1a, w1c, b1r, W2, b2r)
    return _apply_bn(out_pre, psum, psumsq, gammar, betar)


# speedup vs baseline: 2.9996x; 1.0543x over previous
import functools

import jax
import jax.numpy as jnp
from jax import lax
from jax.experimental import pallas as pl
from jax.experimental.pallas import tpu as pltpu
from jax.experimental.pallas import tpu_sc as plsc

N, M, F, D_E, H = 10000, 32, 128, 16, 128
BN_BLK = 400
NBLK = N // BN_BLK

def _q_kernel(atom_ref, w1b_ref, q_ref):
    q = jnp.dot(atom_ref[...], w1b_ref[...],
                preferred_element_type=jnp.float32)
    rid = (lax.broadcasted_iota(jnp.int32, (BN_BLK, H), 0)
           + pl.program_id(0) * BN_BLK)
    q_ref[...] = jnp.where(rid == 0, -1e9, q)


def _compute_q(atom, w1b):
    return pl.pallas_call(
        _q_kernel,
        grid=(NBLK,),
        in_specs=[
            pl.BlockSpec((BN_BLK, F), lambda i: (i, 0)),
            pl.BlockSpec((F, H), lambda i: (0, 0)),
        ],
        out_specs=pl.BlockSpec((BN_BLK, H), lambda i: (i, 0)),
        out_shape=jax.ShapeDtypeStruct((N, H), jnp.float32),
    )(atom, w1b)


_NC, _NS = 2, 16
_NW = _NC * _NS
_CHUNK = 400
K_SL = 5
N_SL = N // K_SL
_NCHUNK = N_SL // _CHUNK
NBLK_SL = N_SL // BN_BLK


def _make_gather_body(k):
    def body(q_hbm, idx_hbm, out_hbm, idx_v, rows_v, sem):
        wid = lax.axis_index("s") * _NC + lax.axis_index("c")

        def step(c, _):
            off = c * _CHUNK
            src = wid * N + k * N_SL + off
            dst = wid * N_SL + off
            pltpu.sync_copy(idx_hbm.at[pl.ds(src, _CHUNK)], idx_v)
            pltpu.async_copy(q_hbm.at[idx_v], rows_v, sem).wait()
            pltpu.sync_copy(rows_v, out_hbm.at[pl.ds(dst, _CHUNK)])
            return ()

        lax.fori_loop(0, _NCHUNK, step, (), unroll=False)
    return body


def _gather_rows(q, idx_flat_t, k):
    mesh = plsc.VectorSubcoreMesh(core_axis_name="c", subcore_axis_name="s")
    fn = functools.partial(
        pl.kernel, mesh=mesh,
        out_type=jax.ShapeDtypeStruct((M * N_SL, H), jnp.float32),
        scratch_types=[
            pltpu.VMEM((_CHUNK,), jnp.int32),
            pltpu.VMEM((_CHUNK, H), jnp.float32),
            pltpu.SemaphoreType.DMA,
        ],
    )(_make_gather_body(k))
    return fn(q, idx_flat_t)


_LOG2E = 1.4426950408889634


def _softplus(x):
    t = lax.exp2(-jnp.abs(x) * _LOG2E)
    return jnp.maximum(x, 0.0) + jnp.log(1.0 + t)


def _msg_kernel(atom_ref, qg_ref, nbr_ref, idxf_ref, w1a_ref, w1c_ref,
                b1_ref, w2_ref, b2_ref, out_ref, psum_ref, psumsq_ref):
    atom = atom_ref[...]
    p = jnp.dot(atom, w1a_ref[...],
                preferred_element_type=jnp.float32) + b1_ref[...]
    w1c = w1c_ref[...]
    acc = None
    for j in range(M):
        ej = jnp.dot(nbr_ref[:, j * D_E:(j + 1) * D_E], w1c,
                     preferred_element_type=jnp.float32)
        hj = _softplus(qg_ref[j] + ej + p)
        acc = hj if acc is None else acc + hj
    cnt = jnp.sum(jnp.where(idxf_ref[...] != 0.0, 1.0, 0.0),
                  axis=1, keepdims=True)
    msg = (jnp.dot(acc, w2_ref[...], preferred_element_type=jnp.float32)
           + cnt * b2_ref[...])
    out_pre = atom + msg
    out_ref[...] = out_pre
    psum_ref[...] = jnp.sum(out_pre, axis=0, keepdims=True)[None]
    psumsq_ref[...] = jnp.sum(out_pre * out_pre, axis=0, keepdims=True)[None]


def _compute_msg(atom, qg3, nbr2, idxf2, w1a, w1c, b1r, w2, b2r, k):
    blk0 = k * NBLK_SL
    return pl.pallas_call(
        _msg_kernel,
        grid=(NBLK_SL,),
        in_specs=[
            pl.BlockSpec((BN_BLK, F), lambda i: (blk0 + i, 0)),
            pl.BlockSpec((M, BN_BLK, H), lambda i: (0, i, 0)),
            pl.BlockSpec((BN_BLK, M * D_E), lambda i: (blk0 + i, 0)),
            pl.BlockSpec((BN_BLK, M), lambda i: (blk0 + i, 0)),
            pl.BlockSpec((F, H), lambda i: (0, 0)),
            pl.BlockSpec((D_E, H), lambda i: (0, 0)),
            pl.BlockSpec((1, H), lambda i: (0, 0)),
            pl.BlockSpec((H, F), lambda i: (0, 0)),
            pl.BlockSpec((1, F), lambda i: (0, 0)),
        ],
        out_specs=[
            pl.BlockSpec((BN_BLK, F), lambda i: (i, 0)),
            pl.BlockSpec((1, 1, F), lambda i: (i, 0, 0)),
            pl.BlockSpec((1, 1, F), lambda i: (i, 0, 0)),
        ],
        out_shape=[
            jax.ShapeDtypeStruct((N_SL, F), jnp.float32),
            jax.ShapeDtypeStruct((NBLK_SL, 1, F), jnp.float32),
            jax.ShapeDtypeStruct((NBLK_SL, 1, F), jnp.float32),
        ],
    )(atom, qg3, nbr2, idxf2, w1a, w1c, b1r, w2, b2r)


def _bn_kernel(x_ref, psum_ref, psumsq_ref, gamma_ref, beta_ref, out_ref):
    mean = jnp.sum(psum_ref[...], axis=0) / N
    ex2 = jnp.sum(psumsq_ref[...], axis=0) / N
    var = ex2 - mean * mean
    inv = lax.rsqrt(var + 1e-5)
    y = (x_ref[...] - mean) * (inv * gamma_ref[...]) + beta_ref[...]
    out_ref[...] = _softplus(y)


def _apply_bn(x, psum, psumsq, gammar, betar):
    return pl.pallas_call(
        _bn_kernel,
        grid=(NBLK,),
        in_specs=[
            pl.BlockSpec((BN_BLK, F), lambda i: (i, 0)),
            pl.BlockSpec((NBLK, 1, F), lambda i: (0, 0, 0)),
            pl.BlockSpec((NBLK, 1, F), lambda i: (0, 0, 0)),
            pl.BlockSpec((1, F), lambda i: (0, 0)),
            pl.BlockSpec((1, F), lambda i: (0, 0)),
        ],
        out_specs=pl.BlockSpec((BN_BLK, F), lambda i: (i, 0)),
        out_shape=jax.ShapeDtypeStruct((N, F), jnp.float32),
    )(x, psum, psumsq, gammar, betar)


def kernel(atom_in_fea, nbr_fea, nbr_fea_idx, W1, b1, W2, b2,
           bn_gamma, bn_beta):
    w1a = W1[:F]
    w1b = W1[F:2 * F]
    w1c = W1[2 * F:]
    b1r = b1.reshape(1, H)
    b2r = b2.reshape(1, F)
    gammar = bn_gamma.reshape(1, F)
    betar = bn_beta.reshape(1, F)
    nbr2 = nbr_fea.reshape(N, M * D_E)
    idxf2 = nbr_fea_idx.astype(jnp.float32)

    idx_flat_t = nbr_fea_idx.T.reshape(M * N)
    q = _compute_q(atom_in_fea, w1b)

    outs, psums, psumsqs = [], [], []
    for k in range(K_SL):
        qg3 = _gather_rows(q, idx_flat_t, k).reshape(M, N_SL, H)
        o, ps, pq = _compute_msg(
            atom_in_fea, qg3, nbr2, idxf2, w1a, w1c, b1r, W2, b2r, k)
        outs.append(o)
        psums.append(ps)
        psumsqs.append(pq)

    out_pre = jnp.concatenate(outs, axis=0)
    psum = jnp.concatenate(psums, axis=0)
    psumsq = jnp.concatenate(psumsqs, axis=0)
    return _apply_bn(out_pre, psum, psumsq, gammar, betar)
